# Initial kernel scaffold; baseline (speedup 1.0000x reference)
import jax
import jax.numpy as jnp
from jax.experimental import pallas as pl


def kernel(alpha, doc_bow, topic_embeddings_mat, word_embeddings_mat):
    raise NotImplementedError("write your pallas kernel here")


# speedup vs baseline: 1.3449x; 1.3449x over previous
import functools

import jax
import jax.numpy as jnp
from jax import lax
from jax.experimental import pallas as pl
from jax.experimental.pallas import tpu as pltpu
from jax.experimental.pallas import tpu_sc as plsc

B = 1024
K = 128
E = 256
V = 100000
TOPK = 25
VT = 2048
NT = 49
VP = NT * VT
G = 128
NG = VP // G
GPT = VT // G
M = K * TOPK


def _k1_body(te_ref, we_ref, e_ref, gmax_ref, z_ref):
    i = pl.program_id(0)
    wt = jnp.dot(te_ref[...], we_ref[...], preferred_element_type=jnp.float32)
    col = i * VT + lax.broadcasted_iota(jnp.int32, (K, VT), 1)
    e = jnp.where(col < V, jnp.exp(wt), 0.0)
    e_ref[...] = e
    gi = lax.broadcasted_iota(jnp.int32, (K, GPT), 1)
    gm = jnp.zeros((K, GPT), jnp.float32)
    for g in range(GPT):
        m = jnp.max(e[:, g * G:(g + 1) * G], axis=1, keepdims=True)
        gm = jnp.where(gi == g, m, gm)
    gmax_ref[...] = gm[None]
    zp = jnp.sum(e, axis=1, keepdims=True)

    @pl.when(i == 0)
    def _():
        z_ref[...] = zp

    @pl.when(i > 0)
    def _():
        z_ref[...] = z_ref[...] + zp


def _k2_body(gm_ref, gid_ref):
    g = gm_ref[...]
    gi = lax.broadcasted_iota(jnp.int32, (K, NG), 1)
    oi = lax.broadcasted_iota(jnp.int32, (K, TOPK), 1)
    gid = jnp.zeros((K, TOPK), jnp.int32)
    for j in range(TOPK):
        m = jnp.max(g, axis=1, keepdims=True)
        pos = jnp.min(jnp.where(g == m, gi, 2**30), axis=1, keepdims=True)
        gid = jnp.where(oi == j, pos, gid)
        g = jnp.where(gi == pos, -1.0, g)
    gid_ref[...] = gid


def _k4a_body(cand_ref, cols_ref, s_ref, p_ref):
    c = cand_ref[...]
    colv = cols_ref[...]
    oi = lax.broadcasted_iota(jnp.int32, (K, TOPK), 1)
    vals = jnp.zeros((K, TOPK), jnp.float32)
    poss = jnp.zeros((K, TOPK), jnp.int32)
    for j in range(TOPK):
        m = jnp.max(c, axis=1, keepdims=True)
        pos = jnp.min(jnp.where(c == m, colv, 2**30), axis=1, keepdims=True)
        vals = jnp.where(oi == j, m, vals)
        poss = jnp.where(oi == j, pos, poss)
        c = jnp.where(colv == pos, -1.0, c)
    s_ref[...] = vals / jnp.sum(vals, axis=1, keepdims=True)
    p_ref[...] = poss


def _k4b_body(s_ref, p_ref, sf_ref, pf_ref, out_ref):
    S = s_ref[...]
    Pp = p_ref[...]
    sf = sf_ref[...]
    pf = pf_ref[...]
    acc = jnp.zeros((K, M), jnp.float32)
    for a in range(TOPK):
        pa = Pp[:, a:a + 1]
        sa = S[:, a:a + 1]
        acc = acc + jnp.where(pf == pa, sa * sf, 0.0)
    r = (lax.broadcasted_iota(jnp.int32, (M, K), 0) // TOPK ==
         lax.broadcasted_iota(jnp.int32, (M, K), 1)).astype(jnp.float32)
    gram = jnp.dot(acc, r, preferred_element_type=jnp.float32)
    ncol = jnp.sum(S * S, axis=1, keepdims=True)
    nrow = jnp.dot(sf * sf, r, preferred_element_type=jnp.float32)
    d = jnp.maximum(ncol + nrow - 2.0 * gram, 0.0)
    t = jnp.sum(d, axis=1, keepdims=True)
    out_ref[...] = 0.5 * jnp.sum(t, axis=0, keepdims=True) / (K * K)


def _k5_body(alpha_ref, z_ref, e_ref, db_ref, re_ref, th_ref):
    i = pl.program_id(0)
    a = alpha_ref[...]
    mx = jnp.max(a, axis=1, keepdims=True)
    ex = jnp.exp(a - mx)
    th = ex / jnp.sum(ex, axis=1, keepdims=True)

    @pl.when(i == 0)
    def _():
        th_ref[...] = th

    tp = th * (1.0 / z_ref[...])
    p = jnp.dot(tp, e_ref[...], preferred_element_type=jnp.float32)
    col = i * VT + lax.broadcasted_iota(jnp.int32, (B, VT), 1)
    valid = col < V
    lg = jnp.log(jnp.where(valid, p, 1.0))
    contrib = lg * jnp.where(valid, db_ref[...], 0.0)
    acc = jnp.sum(contrib, axis=1, keepdims=True)

    @pl.when(i == 0)
    def _():
        re_ref[...] = -acc

    @pl.when(i > 0)
    def _():
        re_ref[...] = re_ref[...] - acc


def _sc_gather(table, idx, nrows, nc):
    nw = nc * 16
    bpw = nrows // nw
    mesh = plsc.VectorSubcoreMesh(core_axis_name="c", subcore_axis_name="s")

    @functools.partial(
        pl.kernel, mesh=mesh,
        out_type=jax.ShapeDtypeStruct((nrows, G), jnp.float32),
        scratch_types=[
            pltpu.VMEM((bpw,), jnp.int32),
            pltpu.VMEM((bpw, G), jnp.float32),
            pltpu.SemaphoreType.DMA,
        ],
    )
    def k(table_hbm, idx_hbm, out_hbm, idx_v, rows_v, sem):
        wid = lax.axis_index("s") * nc + lax.axis_index("c")
        base = wid * bpw
        pltpu.sync_copy(idx_hbm.at[pl.ds(base, bpw)], idx_v)
        pltpu.async_copy(table_hbm.at[idx_v], rows_v, sem).wait()
        pltpu.sync_copy(rows_v, out_hbm.at[pl.ds(base, bpw)])

    return k(table, idx)


def kernel(alpha, doc_bow, topic_embeddings_mat, word_embeddings_mat):
    te = topic_embeddings_mat
    we = word_embeddings_mat

    e, gmax3, z = pl.pallas_call(
        _k1_body,
        grid=(NT,),
        in_specs=[
            pl.BlockSpec((K, E), lambda i: (0, 0)),
            pl.BlockSpec((E, VT), lambda i: (0, i)),
        ],
        out_specs=[
            pl.BlockSpec((K, VT), lambda i: (0, i)),
            pl.BlockSpec((1, K, GPT), lambda i: (i, 0, 0)),
            pl.BlockSpec((K, 1), lambda i: (0, 0)),
        ],
        out_shape=[
            jax.ShapeDtypeStruct((K, VP), jnp.float32),
            jax.ShapeDtypeStruct((NT, K, GPT), jnp.float32),
            jax.ShapeDtypeStruct((K, 1), jnp.float32),
        ],
    )(te, we)

    gmax = gmax3.transpose(1, 0, 2).reshape(K, NG)
    gid = pl.pallas_call(
        _k2_body,
        out_shape=jax.ShapeDtypeStruct((K, TOPK), jnp.int32),
    )(gmax)

    info = plsc.get_sparse_core_info()
    nc = info.num_cores
    align = 8 * nc * 16
    nrows = ((M + align - 1) // align) * align
    rowid = jnp.arange(K, dtype=jnp.int32)[:, None] * NG + gid
    idx = jnp.concatenate(
        [rowid.reshape(-1), jnp.zeros((nrows - M,), jnp.int32)])
    table = e.reshape(K * NG, G)
    cand = _sc_gather(table, idx, nrows, nc)[:M].reshape(K, TOPK * G)
    cols = (gid[:, :, None] * G +
            jnp.arange(G, dtype=jnp.int32)[None, None, :]).reshape(K, TOPK * G)

    S, poss = pl.pallas_call(
        _k4a_body,
        out_shape=[
            jax.ShapeDtypeStruct((K, TOPK), jnp.float32),
            jax.ShapeDtypeStruct((K, TOPK), jnp.int32),
        ],
    )(cand, cols)

    stdr = pl.pallas_call(
        _k4b_body,
        out_shape=jax.ShapeDtypeStruct((1, 1), jnp.float32),
    )(S, poss, S.reshape(1, M), poss.reshape(1, M))

    zrow = z.reshape(1, K)
    re, theta = pl.pallas_call(
        _k5_body,
        grid=(NT,),
        in_specs=[
            pl.BlockSpec((B, K), lambda i: (0, 0)),
            pl.BlockSpec((1, K), lambda i: (0, 0)),
            pl.BlockSpec((K, VT), lambda i: (0, i)),
            pl.BlockSpec((B, VT), lambda i: (0, i)),
        ],
        out_specs=[
            pl.BlockSpec((B, 1), lambda i: (0, 0)),
            pl.BlockSpec((B, K), lambda i: (0, 0)),
        ],
        out_shape=[
            jax.ShapeDtypeStruct((B, 1), jnp.float32),
            jax.ShapeDtypeStruct((B, K), jnp.float32),
        ],
    )(alpha, zrow, e, doc_bow)

    return (re.reshape(B), stdr.reshape(()), theta)


# speedup vs baseline: 1.3678x; 1.0171x over previous
import functools

import jax
import jax.numpy as jnp
from jax import lax
from jax.experimental import pallas as pl
from jax.experimental.pallas import tpu as pltpu
from jax.experimental.pallas import tpu_sc as plsc

B = 1024
K = 128
E = 256
V = 100000
TOPK = 25
VT = 2048
NT = 49
VP = NT * VT
G = 128
NG = VP // G
GPT = VT // G
M = K * TOPK


def _k1_body(te_ref, we_ref, e_ref, gmax_ref, z_ref):
    i = pl.program_id(0)
    wt = jnp.dot(te_ref[...], we_ref[...], preferred_element_type=jnp.float32)
    col = i * VT + lax.broadcasted_iota(jnp.int32, (K, VT), 1)
    e = jnp.where(col < V, jnp.exp(wt), 0.0)
    gi = lax.broadcasted_iota(jnp.int32, (K, GPT), 1)
    gm = jnp.zeros((K, GPT), jnp.float32)
    for g in range(GPT):
        seg = e[:, g * G:(g + 1) * G]
        e_ref[:, g, :] = seg
        m = jnp.max(seg, axis=1, keepdims=True)
        gm = jnp.where(gi == g, m, gm)
    gmax_ref[...] = gm[None]
    zp = jnp.sum(e, axis=1, keepdims=True)

    @pl.when(i == 0)
    def _():
        z_ref[...] = zp

    @pl.when(i > 0)
    def _():
        z_ref[...] = z_ref[...] + zp


def _k2_body(gm_ref, gid_ref):
    g = gm_ref[...]
    gi = lax.broadcasted_iota(jnp.int32, (K, NG), 1)
    oi = lax.broadcasted_iota(jnp.int32, (K, TOPK), 1)
    gid = jnp.zeros((K, TOPK), jnp.int32)
    for j in range(TOPK):
        m = jnp.max(g, axis=1, keepdims=True)
        pos = jnp.min(jnp.where(g == m, gi, 2**30), axis=1, keepdims=True)
        gid = jnp.where(oi == j, pos, gid)
        g = jnp.where(gi == pos, -1.0, g)
    gid_ref[...] = gid


def _k4a_body(cand_ref, cols_ref, s_ref, p_ref):
    c = cand_ref[...]
    colv = cols_ref[...]
    oi = lax.broadcasted_iota(jnp.int32, (K, TOPK), 1)
    vals = jnp.zeros((K, TOPK), jnp.float32)
    poss = jnp.zeros((K, TOPK), jnp.int32)
    for j in range(TOPK):
        m = jnp.max(c, axis=1, keepdims=True)
        pos = jnp.min(jnp.where(c == m, colv, 2**30), axis=1, keepdims=True)
        vals = jnp.where(oi == j, m, vals)
        poss = jnp.where(oi == j, pos, poss)
        c = jnp.where(colv == pos, -1.0, c)
    s_ref[...] = vals / jnp.sum(vals, axis=1, keepdims=True)
    p_ref[...] = poss


def _k4b_body(s_ref, p_ref, sf_ref, pf_ref, out_ref):
    S = s_ref[...]
    Pp = p_ref[...]
    sf = sf_ref[...]
    pf = pf_ref[...]
    acc = jnp.zeros((K, M), jnp.float32)
    for a in range(TOPK):
        pa = Pp[:, a:a + 1]
        sa = S[:, a:a + 1]
        acc = acc + jnp.where(pf == pa, sa * sf, 0.0)
    r = (lax.broadcasted_iota(jnp.int32, (M, K), 0) // TOPK ==
         lax.broadcasted_iota(jnp.int32, (M, K), 1)).astype(jnp.float32)
    gram = jnp.dot(acc, r, preferred_element_type=jnp.float32)
    ncol = jnp.sum(S * S, axis=1, keepdims=True)
    nrow = jnp.dot(sf * sf, r, preferred_element_type=jnp.float32)
    d = jnp.maximum(ncol + nrow - 2.0 * gram, 0.0)
    t = jnp.sum(d, axis=1, keepdims=True)
    out_ref[...] = 0.5 * jnp.sum(t, axis=0, keepdims=True) / (K * K)


def _k5_body(alpha_ref, z_ref, e_ref, db_ref, re_ref, th_ref):
    i = pl.program_id(0)
    a = alpha_ref[...]
    mx = jnp.max(a, axis=1, keepdims=True)
    ex = jnp.exp(a - mx)
    th = ex / jnp.sum(ex, axis=1, keepdims=True)

    @pl.when(i == 0)
    def _():
        th_ref[...] = th

    tp = th * (1.0 / z_ref[...])
    ev = jnp.concatenate([e_ref[:, g, :] for g in range(GPT)], axis=1)
    p = jnp.dot(tp, ev, preferred_element_type=jnp.float32)
    col = i * VT + lax.broadcasted_iota(jnp.int32, (B, VT), 1)
    valid = col < V
    lg = jnp.log(jnp.where(valid, p, 1.0))
    contrib = lg * jnp.where(valid, db_ref[...], 0.0)
    acc = jnp.sum(contrib, axis=1, keepdims=True)

    @pl.when(i == 0)
    def _():
        re_ref[...] = -acc

    @pl.when(i > 0)
    def _():
        re_ref[...] = re_ref[...] - acc


def _sc_gather(table, idx, nrows, nc):
    nw = nc * 16
    bpw = nrows // nw
    mesh = plsc.VectorSubcoreMesh(core_axis_name="c", subcore_axis_name="s")

    @functools.partial(
        pl.kernel, mesh=mesh,
        out_type=jax.ShapeDtypeStruct((nrows, G), jnp.float32),
        scratch_types=[
            pltpu.VMEM((bpw,), jnp.int32),
            pltpu.VMEM((bpw, G), jnp.float32),
            pltpu.SemaphoreType.DMA,
        ],
    )
    def k(table_hbm, idx_hbm, out_hbm, idx_v, rows_v, sem):
        wid = lax.axis_index("s") * nc + lax.axis_index("c")
        base = wid * bpw
        pltpu.sync_copy(idx_hbm.at[pl.ds(base, bpw)], idx_v)
        pltpu.async_copy(table_hbm.at[idx_v], rows_v, sem).wait()
        pltpu.sync_copy(rows_v, out_hbm.at[pl.ds(base, bpw)])

    return k(table, idx)


def kernel(alpha, doc_bow, topic_embeddings_mat, word_embeddings_mat):
    te = topic_embeddings_mat
    we = word_embeddings_mat

    e, gmax3, z = pl.pallas_call(
        _k1_body,
        grid=(NT,),
        in_specs=[
            pl.BlockSpec((K, E), lambda i: (0, 0)),
            pl.BlockSpec((E, VT), lambda i: (0, i)),
        ],
        out_specs=[
            pl.BlockSpec((K, GPT, G), lambda i: (0, i, 0)),
            pl.BlockSpec((1, K, GPT), lambda i: (i, 0, 0)),
            pl.BlockSpec((K, 1), lambda i: (0, 0)),
        ],
        out_shape=[
            jax.ShapeDtypeStruct((K, NG, G), jnp.float32),
            jax.ShapeDtypeStruct((NT, K, GPT), jnp.float32),
            jax.ShapeDtypeStruct((K, 1), jnp.float32),
        ],
    )(te, we)

    gmax = gmax3.transpose(1, 0, 2).reshape(K, NG)
    gid = pl.pallas_call(
        _k2_body,
        out_shape=jax.ShapeDtypeStruct((K, TOPK), jnp.int32),
    )(gmax)

    info = plsc.get_sparse_core_info()
    nc = info.num_cores
    align = 8 * nc * 16
    nrows = ((M + align - 1) // align) * align
    rowid = jnp.arange(K, dtype=jnp.int32)[:, None] * NG + gid
    idx = jnp.concatenate(
        [rowid.reshape(-1), jnp.zeros((nrows - M,), jnp.int32)])
    table = e.reshape(K * NG, G)
    cand = _sc_gather(table, idx, nrows, nc)[:M].reshape(K, TOPK * G)
    cols = (gid[:, :, None] * G +
            jnp.arange(G, dtype=jnp.int32)[None, None, :]).reshape(K, TOPK * G)

    S, poss = pl.pallas_call(
        _k4a_body,
        out_shape=[
            jax.ShapeDtypeStruct((K, TOPK), jnp.float32),
            jax.ShapeDtypeStruct((K, TOPK), jnp.int32),
        ],
    )(cand, cols)

    stdr = pl.pallas_call(
        _k4b_body,
        out_shape=jax.ShapeDtypeStruct((1, 1), jnp.float32),
    )(S, poss, S.reshape(1, M), poss.reshape(1, M))

    zrow = z.reshape(1, K)
    re, theta = pl.pallas_call(
        _k5_body,
        grid=(NT,),
        in_specs=[
            pl.BlockSpec((B, K), lambda i: (0, 0)),
            pl.BlockSpec((1, K), lambda i: (0, 0)),
            pl.BlockSpec((K, GPT, G), lambda i: (0, i, 0)),
            pl.BlockSpec((B, VT), lambda i: (0, i)),
        ],
        out_specs=[
            pl.BlockSpec((B, 1), lambda i: (0, 0)),
            pl.BlockSpec((B, K), lambda i: (0, 0)),
        ],
        out_shape=[
            jax.ShapeDtypeStruct((B, 1), jnp.float32),
            jax.ShapeDtypeStruct((B, K), jnp.float32),
        ],
    )(alpha, zrow, e, doc_bow)

    return (re.reshape(B), stdr.reshape(()), theta)
